# Initial kernel scaffold; baseline (speedup 1.0000x reference)
import jax
import jax.numpy as jnp
from jax.experimental import pallas as pl


def kernel(scores_w, scores_h, bbox_deltas_w, bbox_deltas_h, im_info):
    raise NotImplementedError("write your pallas kernel here")


# speedup vs baseline: 58.8646x; 58.8646x over previous
import jax
import jax.numpy as jnp
import numpy as np
from jax.experimental import pallas as pl
from jax.experimental.pallas import tpu as pltpu

_FEAT_STRIDE = 16
_W_AN = np.array([16., 32., 64., 128., 256., 512., 1024.], dtype=np.float32)
_H_AN = np.array([16., 32., 64., 128., 256., 512., 1024.], dtype=np.float32)
_PRE_NMS_TOPN = 6000
_POST_NMS_TOPN = 300
_NMS_THRESH = 0.7
_COM_TOPN = 2000
_COM_TOPK = 3

_BLK = 128
_NBLK = 47
_NPAD = _BLK * _NBLK
_RANKS = 304


def _anchor_strings(w):
    return np.stack([-(w - 1.0) / 2.0, (w - 1.0) / 2.0], axis=1).astype(np.float32)


_ANCH_W = _anchor_strings(_W_AN)
_ANCH_H = _anchor_strings(_H_AN)


def _nms_kernel(rows_ref, cols_ref, out_ref, keep_ref, m_ref, cnt_ref):
    out_ref[...] = jnp.zeros_like(out_ref)
    cnt_ref[0] = 0

    lane_i = jax.lax.broadcasted_iota(jnp.int32, (1, _BLK), 1)
    sub2 = jax.lax.broadcasted_iota(jnp.int32, (_BLK, _BLK), 0)
    lan2 = jax.lax.broadcasted_iota(jnp.int32, (_BLK, _BLK), 1)
    tri = (sub2 < lan2).astype(jnp.float32)
    csum_mat = (sub2 <= lan2).astype(jnp.float32)
    rho = jax.lax.broadcasted_iota(
        jnp.int32, (_RANKS, _BLK), 0).astype(jnp.float32)

    def row_coords(j):
        v = rows_ref[0, j, :, :]
        x1, y1, x2, y2 = v[0:1, :], v[1:2, :], v[2:3, :], v[3:4, :]
        ar = (x2 - x1 + 1.0) * (y2 - y1 + 1.0)
        return x1, y1, x2, y2, ar

    def col_coords(i):
        v = cols_ref[0, i, :, :]
        x1, y1, x2, y2 = v[:, 0:1], v[:, 1:2], v[:, 2:3], v[:, 3:4]
        ar = (x2 - x1 + 1.0) * (y2 - y1 + 1.0)
        return x1, y1, x2, y2, ar

    def pair_sup(c, r):
        cx1, cy1, cx2, cy2, car = c
        rx1, ry1, rx2, ry2, rar = r
        xx1 = jnp.maximum(cx1, rx1)
        yy1 = jnp.maximum(cy1, ry1)
        xx2 = jnp.minimum(cx2, rx2)
        yy2 = jnp.minimum(cy2, ry2)
        iw = jnp.maximum(0.0, xx2 - xx1 + 1.0)
        ih = jnp.maximum(0.0, yy2 - yy1 + 1.0)
        inter = iw * ih
        iou = inter / (car + rar - inter)
        return (iou > _NMS_THRESH).astype(jnp.float32)

    def block_body(j, carry):
        @pl.when(cnt_ref[0] < _POST_NMS_TOPN)
        def _():
            r = row_coords(j)
            m_ref[...] = pair_sup(col_coords(j), r) * tri

            def cross(i, sup):
                mp = pair_sup(col_coords(i), r)
                ki = keep_ref[pl.ds(i, 1), :]
                s = jax.lax.dot(ki, mp, preferred_element_type=jnp.float32)
                return jnp.maximum(sup, s)

            sup0 = jax.lax.fori_loop(0, j, cross,
                                     jnp.zeros((1, _BLK), jnp.float32))

            def step(t, sup):
                e = lane_i == t
                s_t = jnp.max(jnp.where(e, sup, 0.0), axis=1, keepdims=True)
                gate = jnp.where(s_t > 0.0, 0.0, 1.0)
                row_t = m_ref[pl.ds(t, 1), :]
                return jnp.maximum(sup, row_t * gate)

            sup = jax.lax.fori_loop(0, _BLK, step, sup0)
            keep = jnp.where(sup > 0.0, 0.0, 1.0)
            keep_ref[pl.ds(j, 1), :] = keep

            csum = jax.lax.dot(keep, csum_mat,
                               preferred_element_type=jnp.float32)
            cntf = cnt_ref[0].astype(jnp.float32)
            ranks = jnp.where(keep > 0.0, cntf + csum - 1.0, -1.0)
            p = (rho == ranks).astype(jnp.float32)
            bt = cols_ref[0, j, :, :]
            out_ref[0] = out_ref[0] + jax.lax.dot(
                p, bt, preferred_element_type=jnp.float32)
            cnt_ref[0] = cnt_ref[0] + jnp.sum(keep).astype(jnp.int32)
        return carry

    jax.lax.fori_loop(0, _NBLK, block_body, 0)


def _nms_select(boxes_sorted):
    B = boxes_sorted.shape[0]
    pad = jnp.zeros((B, _NPAD - boxes_sorted.shape[1], 8), jnp.float32)
    b8 = jnp.concatenate(
        [boxes_sorted,
         jnp.zeros((B, boxes_sorted.shape[1], 4), jnp.float32)], axis=2)
    b8 = jnp.concatenate([b8, pad], axis=1)
    cols = b8.reshape(B, _NBLK, _BLK, 8)
    rows = jnp.transpose(b8, (0, 2, 1)).reshape(B, 8, _NBLK, _BLK)
    rows = jnp.transpose(rows, (0, 2, 1, 3))

    out = pl.pallas_call(
        _nms_kernel,
        grid=(B,),
        in_specs=[
            pl.BlockSpec((1, _NBLK, 8, _BLK), lambda b: (b, 0, 0, 0)),
            pl.BlockSpec((1, _NBLK, _BLK, 8), lambda b: (b, 0, 0, 0)),
        ],
        out_specs=pl.BlockSpec((1, _RANKS, 8), lambda b: (b, 0, 0)),
        out_shape=jax.ShapeDtypeStruct((B, _RANKS, 8), jnp.float32),
        scratch_shapes=[
            pltpu.VMEM((48, _BLK), jnp.float32),
            pltpu.VMEM((_BLK, _BLK), jnp.float32),
            pltpu.SMEM((1,), jnp.int32),
        ],
    )(rows, cols)
    return out[:, :_POST_NMS_TOPN, :4]


def _bbox_transform_inv(strings, deltas):
    widths = strings[..., 1] - strings[..., 0] + 1.0
    ctr = strings[..., 0] + 0.5 * widths
    d_ctr = deltas[..., 0]
    d_w = jnp.clip(deltas[..., 1], -10.0, 4.0)
    pred_ctr = d_ctr * widths + ctr
    pred_w = jnp.exp(d_w) * widths
    return jnp.stack([pred_ctr - 0.5 * (pred_w - 1.0),
                      pred_ctr + 0.5 * (pred_w - 1.0)], axis=-1)


def _strings_to_proposals(prop_a, prop_b, sc_a, sc_b, A, K, primary_is_w):
    B = prop_a.shape[0]
    top_sa, top_ia = jax.lax.top_k(sc_a, _COM_TOPN)
    pos = top_ia // A
    a_sel = jnp.take_along_axis(prop_a, top_ia[..., None], axis=1)
    sc_b_r = sc_b.reshape(B, K, A)
    prop_b_r = prop_b.reshape(B, K, A, 2)
    top_sb, top_ib = jax.lax.top_k(sc_b_r, _COM_TOPK)
    b_strings = jnp.take_along_axis(prop_b_r, top_ib[..., None], axis=2)
    idx_s = jnp.broadcast_to(pos[:, :, None], (B, _COM_TOPN, _COM_TOPK))
    sb_at = jnp.take_along_axis(top_sb, idx_s, axis=1)
    idx_b = jnp.broadcast_to(pos[:, :, None, None], (B, _COM_TOPN, _COM_TOPK, 2))
    b_at = jnp.take_along_axis(b_strings, idx_b, axis=1)
    a_exp = jnp.broadcast_to(a_sel[:, :, None, :], (B, _COM_TOPN, _COM_TOPK, 2))
    if primary_is_w:
        boxes = jnp.stack([a_exp[..., 0], b_at[..., 0],
                           a_exp[..., 1], b_at[..., 1]], axis=-1)
    else:
        boxes = jnp.stack([b_at[..., 0], a_exp[..., 0],
                           b_at[..., 1], a_exp[..., 1]], axis=-1)
    scores = top_sa[:, :, None] * sb_at
    return (boxes.reshape(B, _COM_TOPN * _COM_TOPK, 4),
            scores.reshape(B, _COM_TOPN * _COM_TOPK))


def _clip_boxes(boxes, im_info):
    h = im_info[:, 0][:, None]
    w = im_info[:, 1][:, None]
    x1 = jnp.clip(boxes[..., 0], 0.0, w - 1.0)
    y1 = jnp.clip(boxes[..., 1], 0.0, h - 1.0)
    x2 = jnp.clip(boxes[..., 2], 0.0, w - 1.0)
    y2 = jnp.clip(boxes[..., 3], 0.0, h - 1.0)
    return jnp.stack([x1, y1, x2, y2], axis=-1)


def _forward_impl(scores_w, scores_h, bbox_deltas_w, bbox_deltas_h, im_info):
    B = scores_w.shape[0]
    A = _ANCH_W.shape[0]
    H, W = scores_w.shape[2], scores_w.shape[3]
    K = H * W
    sc_w = jnp.transpose(scores_w[:, A:], (0, 2, 3, 1)).reshape(B, -1)
    sc_h = jnp.transpose(scores_h[:, A:], (0, 2, 3, 1)).reshape(B, -1)
    d_w = jnp.transpose(bbox_deltas_w, (0, 2, 3, 1)).reshape(B, -1, 2)
    d_h = jnp.transpose(bbox_deltas_h, (0, 2, 3, 1)).reshape(B, -1, 2)
    sx, sy = jnp.meshgrid(jnp.arange(W, dtype=jnp.float32) * _FEAT_STRIDE,
                          jnp.arange(H, dtype=jnp.float32) * _FEAT_STRIDE)
    shifts_x = sx.ravel()
    shifts_y = sy.ravel()
    anch_w = jnp.broadcast_to(
        (_ANCH_W[None, :, :] + shifts_x[:, None, None]).reshape(1, K * A, 2),
        (B, K * A, 2))
    anch_h = jnp.broadcast_to(
        (_ANCH_H[None, :, :] + shifts_y[:, None, None]).reshape(1, K * A, 2),
        (B, K * A, 2))
    prop_w = _bbox_transform_inv(anch_w, d_w)
    prop_h = _bbox_transform_inv(anch_h, d_h)
    bx_w, s_w = _strings_to_proposals(prop_w, prop_h, sc_w, sc_h, A, K, True)
    bx_h, s_h = _strings_to_proposals(prop_h, prop_w, sc_h, sc_w, A, K, False)
    proposals = jnp.concatenate([bx_w, bx_h], axis=1)
    scores = jnp.concatenate([s_w, s_h], axis=1)
    proposals = _clip_boxes(proposals, im_info)
    n_pre = min(_PRE_NMS_TOPN, scores.shape[1])
    top_s, top_i = jax.lax.top_k(scores, n_pre)
    boxes_sorted = jnp.take_along_axis(proposals, top_i[..., None], axis=1)

    out_boxes = _nms_select(boxes_sorted)
    batch_col = jnp.broadcast_to(
        jnp.arange(B, dtype=out_boxes.dtype)[:, None, None],
        (B, _POST_NMS_TOPN, 1))
    return jnp.concatenate([batch_col, out_boxes], axis=2)


_forward_jitted = jax.jit(_forward_impl)


def kernel(scores_w, scores_h, bbox_deltas_w, bbox_deltas_h, im_info):
    return _forward_jitted(scores_w, scores_h, bbox_deltas_w, bbox_deltas_h,
                           im_info)
